# Initial kernel scaffold; baseline (speedup 1.0000x reference)
import functools

import jax
import jax.numpy as jnp
from jax import lax
from jax.experimental import pallas as pl
from jax.experimental.pallas import tpu as pltpu
from jax.experimental.pallas import tpu_sc as plsc

VOCAB = 1000000
EMBED_DIM = 32
BATCH = 4096
HIST = 200

NC = 2
NS = 16
NW = NC * NS
TOTAL = BATCH * HIST
RPW = TOTAL // NW
IW = 128
IDX_ROWS = RPW // IW
SPC = 10
CH = SPC * IW
NCH = RPW // CH
NBUF = 2


def _emb_body(idx_hbm, table_hbm, out_hbm, idx_v, rows_v, gsems, wsems):
    wid = lax.axis_index("s") * NC + lax.axis_index("c")
    base = wid * RPW

    pltpu.sync_copy(idx_hbm.at[wid], idx_v)

    def fire_gather(c, b):
        for j in range(SPC):
            pltpu.async_copy(
                table_hbm.at[idx_v.at[c * SPC + j]],
                rows_v.at[b].at[pl.ds(j * IW, IW)],
                gsems.at[b],
            )

    def drain_gather(b):
        pltpu.make_async_copy(
            table_hbm.at[pl.ds(0, CH)], rows_v.at[b], gsems.at[b]
        ).wait()

    def fire_write(c, b):
        pltpu.async_copy(
            rows_v.at[b], out_hbm.at[pl.ds(base + c * CH, CH)], wsems.at[b]
        )

    def drain_write(b):
        pltpu.make_async_copy(
            rows_v.at[b], out_hbm.at[pl.ds(base, CH)], wsems.at[b]
        ).wait()

    for b in range(NBUF):
        fire_gather(b, b)

    @pl.loop(0, NCH, step=NBUF)
    def _chunks(g):
        for b in range(NBUF):
            c = g + b
            drain_gather(b)
            fire_write(c, b)
            drain_write(b)

            @pl.when(c + NBUF < NCH)
            def _():
                fire_gather(c + NBUF, b)


@jax.jit
def _emb_call(idx3, table):
    mesh = plsc.VectorSubcoreMesh(core_axis_name="c", subcore_axis_name="s")
    f = pl.kernel(
        _emb_body,
        out_type=jax.ShapeDtypeStruct((TOTAL, EMBED_DIM), jnp.float32),
        mesh=mesh,
        scratch_types=[
            pltpu.VMEM((IDX_ROWS, IW), jnp.int32),
            pltpu.VMEM((NBUF, CH, EMBED_DIM), jnp.float32),
            pltpu.SemaphoreType.DMA((NBUF,)),
            pltpu.SemaphoreType.DMA((NBUF,)),
        ],
    )
    return f(idx3, table)


def kernel(x, table):
    idx3 = x.astype(jnp.int32).reshape(NW, IDX_ROWS, IW)
    flat = _emb_call(idx3, table)
    return flat.reshape(BATCH, HIST, EMBED_DIM)


# speedup vs baseline: 1.4997x; 1.4997x over previous
import functools

import jax
import jax.numpy as jnp
from jax import lax
from jax.experimental import pallas as pl
from jax.experimental.pallas import tpu as pltpu
from jax.experimental.pallas import tpu_sc as plsc

VOCAB = 1000000
EMBED_DIM = 32
BATCH = 4096
HIST = 200

NC = 2
NS = 16
NW = NC * NS
TOTAL = BATCH * HIST
RPW = TOTAL // NW
IW = 128
IDX_ROWS = RPW // IW
SPC = 10
CH = SPC * IW
NCH = RPW // CH
NBUF = 2


def _emb_body(idx_hbm, table_hbm, out_hbm, idx_v, rows_v, gsems, wsems):
    wid = lax.axis_index("s") * NC + lax.axis_index("c")
    base = wid * RPW

    pltpu.sync_copy(idx_hbm.at[wid], idx_v)

    def fire_gather(c, b):
        for j in range(SPC):
            pltpu.async_copy(
                table_hbm.at[idx_v.at[c * SPC + j]],
                rows_v.at[b].at[pl.ds(j * IW, IW)],
                gsems.at[b],
            )

    def drain_gather(b):
        pltpu.make_async_copy(
            table_hbm.at[pl.ds(0, CH)], rows_v.at[b], gsems.at[b]
        ).wait()

    def fire_write(c, b):
        pltpu.async_copy(
            rows_v.at[b], out_hbm.at[pl.ds(base + c * CH, CH)], wsems.at[b]
        )

    def drain_write(b):
        pltpu.make_async_copy(
            rows_v.at[b], out_hbm.at[pl.ds(base, CH)], wsems.at[b]
        ).wait()

    for b in range(NBUF):
        fire_gather(b, b)

    @pl.loop(0, NCH, step=NBUF)
    def _chunks(g):
        for b in range(NBUF):
            c = g + b
            drain_gather(b)
            fire_write(c, b)
            drain_write(b)

            @pl.when(c + NBUF < NCH)
            def _():
                fire_gather(c + NBUF, b)


@jax.jit
def _emb_call(idx3, table):
    mesh = plsc.VectorSubcoreMesh(core_axis_name="c", subcore_axis_name="s")
    f = pl.kernel(
        _emb_body,
        out_type=jax.ShapeDtypeStruct((TOTAL, EMBED_DIM), jnp.float32),
        mesh=mesh,
        scratch_types=[
            pltpu.VMEM((IDX_ROWS, IW), jnp.int32),
            pltpu.VMEM((NBUF, CH, EMBED_DIM), jnp.float32),
            pltpu.SemaphoreType.DMA((NBUF,)),
            pltpu.SemaphoreType.DMA((NBUF,)),
        ],
        compiler_params=pltpu.CompilerParams(use_tc_tiling_on_sc=False),
    )
    return f(idx3, table)


def kernel(x, table):
    idx3 = x.astype(jnp.int32).reshape(NW, IDX_ROWS, IW)
    flat = _emb_call(idx3, table)
    return flat.reshape(BATCH, HIST, EMBED_DIM)


# speedup vs baseline: 1.5008x; 1.0007x over previous
import jax
import jax.numpy as jnp
from jax import lax
from jax.experimental import pallas as pl
from jax.experimental.pallas import tpu as pltpu
from jax.experimental.pallas import tpu_sc as plsc

VOCAB = 1000000
EMBED_DIM = 32
BATCH = 4096
HIST = 200

NC = 2
NS = 16
NW = NC * NS
TOTAL = BATCH * HIST
RPW = TOTAL // NW
CH = 1280
NCH = RPW // CH
NBUF = 2


def _emb_body(idx_hbm, table_hbm, out_hbm, idx_v, rows_v, gsems, wsems):
    wid = lax.axis_index("s") * NC + lax.axis_index("c")
    base = wid * RPW

    pltpu.sync_copy(idx_hbm.at[wid], idx_v)

    def fire_gather(c, b):
        pltpu.async_copy(
            table_hbm.at[idx_v.at[pl.ds(c * CH, CH)]],
            rows_v.at[b],
            gsems.at[b],
        )

    def drain_gather(b):
        pltpu.make_async_copy(
            table_hbm.at[pl.ds(0, CH)], rows_v.at[b], gsems.at[b]
        ).wait()

    def fire_write(c, b):
        pltpu.async_copy(
            rows_v.at[b], out_hbm.at[pl.ds(base + c * CH, CH)], wsems.at[b]
        )

    def drain_write(b):
        pltpu.make_async_copy(
            rows_v.at[b], out_hbm.at[pl.ds(base, CH)], wsems.at[b]
        ).wait()

    for b in range(NBUF):
        fire_gather(b, b)

    @pl.loop(0, NCH, step=NBUF)
    def _chunks(g):
        for b in range(NBUF):
            c = g + b
            drain_gather(b)
            fire_write(c, b)
            drain_write(b)

            @pl.when(c + NBUF < NCH)
            def _():
                fire_gather(c + NBUF, b)


@jax.jit
def _emb_call(idx2, table):
    mesh = plsc.VectorSubcoreMesh(core_axis_name="c", subcore_axis_name="s")
    f = pl.kernel(
        _emb_body,
        out_type=jax.ShapeDtypeStruct((TOTAL, EMBED_DIM), jnp.float32),
        mesh=mesh,
        scratch_types=[
            pltpu.VMEM((RPW,), jnp.int32),
            pltpu.VMEM((NBUF, CH, EMBED_DIM), jnp.float32),
            pltpu.SemaphoreType.DMA((NBUF,)),
            pltpu.SemaphoreType.DMA((NBUF,)),
        ],
        compiler_params=pltpu.CompilerParams(use_tc_tiling_on_sc=False),
    )
    return f(idx2, table)


def kernel(x, table):
    idx2 = x.astype(jnp.int32).reshape(NW, RPW)
    flat = _emb_call(idx2, table)
    return flat.reshape(BATCH, HIST, EMBED_DIM)


# speedup vs baseline: 1.5009x; 1.0001x over previous
import jax
import jax.numpy as jnp
from jax import lax
from jax.experimental import pallas as pl
from jax.experimental.pallas import tpu as pltpu
from jax.experimental.pallas import tpu_sc as plsc

VOCAB = 1000000
EMBED_DIM = 32
BATCH = 4096
HIST = 200

NC = 2
NS = 16
NW = NC * NS
BPW = BATCH // NW
CB = 4
NCH = BPW // CB
NBUF = 2


def _emb_body(x_hbm, table_hbm, out_hbm, idx_v, rows_v, gsems, wsems):
    wid = lax.axis_index("s") * NC + lax.axis_index("c")
    b0 = wid * BPW

    pltpu.sync_copy(x_hbm.at[pl.ds(b0, BPW)], idx_v)

    def fire_gather(c, b):
        for j in range(CB):
            pltpu.async_copy(
                table_hbm.at[idx_v.at[c * CB + j]],
                rows_v.at[b].at[j],
                gsems.at[b],
            )

    def drain_gather(b):
        pltpu.make_async_copy(
            out_hbm.at[pl.ds(0, CB)], rows_v.at[b], gsems.at[b]
        ).wait()

    def fire_write(c, b):
        pltpu.async_copy(
            rows_v.at[b], out_hbm.at[pl.ds(b0 + c * CB, CB)], wsems.at[b]
        )

    def drain_write(b):
        pltpu.make_async_copy(
            rows_v.at[b], out_hbm.at[pl.ds(b0, CB)], wsems.at[b]
        ).wait()

    for b in range(NBUF):
        fire_gather(b, b)

    @pl.loop(0, NCH, step=NBUF)
    def _chunks(g):
        for b in range(NBUF):
            c = g + b
            drain_gather(b)
            fire_write(c, b)
            drain_write(b)

            @pl.when(c + NBUF < NCH)
            def _():
                fire_gather(c + NBUF, b)


@jax.jit
def _emb_call(x, table):
    mesh = plsc.VectorSubcoreMesh(core_axis_name="c", subcore_axis_name="s")
    f = pl.kernel(
        _emb_body,
        out_type=jax.ShapeDtypeStruct((BATCH, HIST, EMBED_DIM), jnp.float32),
        mesh=mesh,
        scratch_types=[
            pltpu.VMEM((BPW, HIST), jnp.int32),
            pltpu.VMEM((NBUF, CB, HIST, EMBED_DIM), jnp.float32),
            pltpu.SemaphoreType.DMA((NBUF,)),
            pltpu.SemaphoreType.DMA((NBUF,)),
        ],
        compiler_params=pltpu.CompilerParams(use_tc_tiling_on_sc=False),
    )
    return f(x, table)


def kernel(x, table):
    return _emb_call(x.astype(jnp.int32), table)
